# baseline (speedup 1.0000x reference)
import functools

import jax
import jax.numpy as jnp
from jax import lax
from jax.experimental import pallas as pl
from jax.experimental.pallas import tpu as pltpu
from jax.experimental.pallas import tpu_sc as plsc

B = 16384
K = 128
NC = 2
NS = 16
NW = NC * NS
ROWS_PER_W = B // NW
CH = 128
NCH = ROWS_PER_W // CH
NTAB = 1000000
NCAT = 2000896


def _sc_diff(eid_hbm, diff_hbm, diff_out, eid_v, dbuf, sem_d, sem_o):
    wid = lax.axis_index("s") * NC + lax.axis_index("c")
    cbase = wid * NCH
    pltpu.sync_copy(eid_hbm.at[pl.ds(cbase, NCH)], eid_v)
    hs = [pltpu.async_copy(diff_hbm.at[eid_v.at[j]], dbuf.at[j], sem_d)
          for j in range(NCH)]
    out_hs = []
    for j in range(NCH):
        hs[j].wait()
        out_hs.append(pltpu.async_copy(
            dbuf.at[j], diff_out.at[pl.ds((cbase + j) * CH, CH)], sem_o))
    for h in out_hs:
        h.wait()


def _sc_blend(sid_hbm, eid_hbm, tab_hbm, s_hbm,
              student_out, disc_out,
              sid_v, eid_v, theta_v, disc_v, s_buf,
              sem_t, sem_c, sem_i, sem_s):
    wid = lax.axis_index("s") * NC + lax.axis_index("c")
    cbase = wid * NCH

    pltpu.sync_copy(sid_hbm.at[pl.ds(cbase, NCH)], sid_v)
    pltpu.sync_copy(eid_hbm.at[pl.ds(cbase, NCH)], eid_v)
    for j in range(NCH):
        for c in range(CH // 16):
            sl = pl.ds(c * 16, 16)
            eid_v[j, sl] = eid_v[j, sl] + NTAB

    hs = []
    for j in range(NCH):
        hs.append(pltpu.async_copy(tab_hbm.at[sid_v.at[j]],
                                   theta_v.at[pl.ds(j * CH, CH)], sem_t))
        hs.append(pltpu.async_copy(tab_hbm.at[eid_v.at[j]], disc_v.at[j],
                                   sem_c))
    h_in0 = pltpu.async_copy(s_hbm.at[pl.ds(cbase * CH, CH)], s_buf.at[0],
                             sem_i)
    for h in hs:
        h.wait()
    pltpu.sync_copy(disc_v, disc_out.at[pl.ds(cbase, NCH)])

    h_in = h_in0
    h_out_prev = None
    for j in range(NCH):
        p = j % 2
        h_in.wait()
        if j + 1 < NCH:
            if h_out_prev is not None:
                h_out_prev.wait()
                h_out_prev = None
            h_in = pltpu.async_copy(
                s_hbm.at[pl.ds((cbase + j + 1) * CH, CH)], s_buf.at[1 - p],
                sem_i)

        def group_body(g, _, j=j, p=p):
            tv = theta_v[pl.ds(j * CH + g * 16, 16)]
            for l in range(16):
                t = tv.at[jnp.full((16,), l, dtype=jnp.int32)].get(
                    mode="promise_in_bounds")
                one_m_t = 1.0 - t
                r = g * 16 + l
                for c in range(K // 16):
                    sl = pl.ds(c * 16, 16)
                    s_buf[p, r, sl] = s_buf[p, r, sl] * one_m_t + t
            return 0

        lax.fori_loop(0, CH // 16, group_body, 0)
        h_out = pltpu.async_copy(
            s_buf.at[p], student_out.at[pl.ds((cbase + j) * CH, CH)], sem_s)
        if h_out_prev is not None:
            h_out_prev.wait()
        h_out_prev = h_out
    h_out_prev.wait()


@jax.jit
def _run(sid2, eid2, S, tab_flat, diff_table):
    mesh = plsc.VectorSubcoreMesh(core_axis_name="c", subcore_axis_name="s")
    diff_ts, = pl.kernel(
        _sc_diff,
        out_type=[jax.ShapeDtypeStruct((B, K), jnp.float32)],
        mesh=mesh,
        scratch_types=[
            pltpu.VMEM((NCH, CH), jnp.int32),
            pltpu.VMEM((NCH, CH, K), jnp.float32),
            pltpu.SemaphoreType.DMA,
            pltpu.SemaphoreType.DMA,
        ],
    )(eid2, diff_table)

    student_ts, disc_g = pl.kernel(
        _sc_blend,
        out_type=[
            jax.ShapeDtypeStruct((B, K), jnp.float32),
            jax.ShapeDtypeStruct((B // CH, CH), jnp.float32),
        ],
        mesh=mesh,
        scratch_types=[
            pltpu.VMEM((NCH, CH), jnp.int32),
            pltpu.VMEM((NCH, CH), jnp.int32),
            pltpu.VMEM((ROWS_PER_W,), jnp.float32),
            pltpu.VMEM((NCH, CH), jnp.float32),
            pltpu.VMEM((2, CH, K), jnp.float32),
            pltpu.SemaphoreType.DMA,
            pltpu.SemaphoreType.DMA,
            pltpu.SemaphoreType.DMA,
            pltpu.SemaphoreType.DMA,
        ],
    )(sid2, eid2, tab_flat, S)

    return student_ts, diff_ts, disc_g


def kernel(student_id, exercise_id, S, theta_tuda, theta, diff_table,
           disc_table):
    sid2 = student_id.reshape(B // CH, CH)
    eid2 = exercise_id.reshape(B // CH, CH)
    tail = jnp.zeros((NCAT - 2 * NTAB, 1), jnp.float32)
    tab_flat = jnp.concatenate([theta, disc_table, tail], axis=0).reshape(-1)
    student_ts, diff_ts, disc_rows = _run(sid2, eid2, S, tab_flat, diff_table)
    return student_ts, diff_ts, disc_rows.reshape(B, 1)


# speedup vs baseline: 3.3228x; 3.3228x over previous
import functools

import jax
import jax.numpy as jnp
from jax import lax
from jax.experimental import pallas as pl
from jax.experimental.pallas import tpu as pltpu
from jax.experimental.pallas import tpu_sc as plsc

B = 16384
K = 128
NC = 2
NS = 16
NW = NC * NS
ROWS_PER_W = B // NW
CH = 128
NCH = ROWS_PER_W // CH
NTAB = 1000000
NPAD = 1000448


def _sc_diff(eid_hbm, diff_hbm, diff_out, eid_v, dbuf, sem_d, sem_o):
    wid = lax.axis_index("s") * NC + lax.axis_index("c")
    cbase = wid * NCH
    pltpu.sync_copy(eid_hbm.at[pl.ds(cbase, NCH)], eid_v)
    hs = [pltpu.async_copy(diff_hbm.at[eid_v.at[j]], dbuf.at[j], sem_d)
          for j in range(NCH)]
    out_hs = []
    for j in range(NCH):
        hs[j].wait()
        out_hs.append(pltpu.async_copy(
            dbuf.at[j], diff_out.at[pl.ds((cbase + j) * CH, CH)], sem_o))
    for h in out_hs:
        h.wait()


def _sc_blend(sid_hbm, eid_hbm, theta_hbm, disc_hbm, s_hbm,
              student_out, disc_out,
              sid_v, eid_v, theta_v, disc_v, s_buf,
              sem_t, sem_c, sem_i, sem_s):
    wid = lax.axis_index("s") * NC + lax.axis_index("c")
    cbase = wid * NCH

    pltpu.sync_copy(sid_hbm.at[pl.ds(cbase, NCH)], sid_v)
    pltpu.sync_copy(eid_hbm.at[pl.ds(cbase, NCH)], eid_v)

    hs = []
    for j in range(NCH):
        hs.append(pltpu.async_copy(theta_hbm.at[sid_v.at[j]],
                                   theta_v.at[pl.ds(j * CH, CH)], sem_t))
        hs.append(pltpu.async_copy(disc_hbm.at[eid_v.at[j]], disc_v.at[j],
                                   sem_c))
    h_in0 = pltpu.async_copy(s_hbm.at[pl.ds(cbase * CH, CH)], s_buf.at[0],
                             sem_i)
    for h in hs:
        h.wait()
    pltpu.sync_copy(disc_v, disc_out.at[pl.ds(cbase, NCH)])

    h_in = h_in0
    h_out_prev = None
    for j in range(NCH):
        p = j % 2
        h_in.wait()
        if j + 1 < NCH:
            if h_out_prev is not None:
                h_out_prev.wait()
                h_out_prev = None
            h_in = pltpu.async_copy(
                s_hbm.at[pl.ds((cbase + j + 1) * CH, CH)], s_buf.at[1 - p],
                sem_i)

        def group_body(g, _, j=j, p=p):
            tv = theta_v[pl.ds(j * CH + g * 16, 16)]
            for l in range(16):
                t = tv.at[jnp.full((16,), l, dtype=jnp.int32)].get(
                    mode="promise_in_bounds")
                one_m_t = 1.0 - t
                r = g * 16 + l
                for c in range(K // 16):
                    sl = pl.ds(c * 16, 16)
                    s_buf[p, r, sl] = s_buf[p, r, sl] * one_m_t + t
            return 0

        lax.fori_loop(0, CH // 16, group_body, 0)
        h_out = pltpu.async_copy(
            s_buf.at[p], student_out.at[pl.ds((cbase + j) * CH, CH)], sem_s)
        if h_out_prev is not None:
            h_out_prev.wait()
        h_out_prev = h_out
    h_out_prev.wait()


@jax.jit
def _run(sid2, eid2, S, theta_flat, disc_flat, diff_table):
    mesh = plsc.VectorSubcoreMesh(core_axis_name="c", subcore_axis_name="s")
    diff_ts, = pl.kernel(
        _sc_diff,
        out_type=[jax.ShapeDtypeStruct((B, K), jnp.float32)],
        mesh=mesh,
        scratch_types=[
            pltpu.VMEM((NCH, CH), jnp.int32),
            pltpu.VMEM((NCH, CH, K), jnp.float32),
            pltpu.SemaphoreType.DMA,
            pltpu.SemaphoreType.DMA,
        ],
    )(eid2, diff_table)

    student_ts, disc_g = pl.kernel(
        _sc_blend,
        out_type=[
            jax.ShapeDtypeStruct((B, K), jnp.float32),
            jax.ShapeDtypeStruct((B // CH, CH), jnp.float32),
        ],
        mesh=mesh,
        scratch_types=[
            pltpu.VMEM((NCH, CH), jnp.int32),
            pltpu.VMEM((NCH, CH), jnp.int32),
            pltpu.VMEM((ROWS_PER_W,), jnp.float32),
            pltpu.VMEM((NCH, CH), jnp.float32),
            pltpu.VMEM((2, CH, K), jnp.float32),
            pltpu.SemaphoreType.DMA,
            pltpu.SemaphoreType.DMA,
            pltpu.SemaphoreType.DMA,
            pltpu.SemaphoreType.DMA,
        ],
    )(sid2, eid2, theta_flat, disc_flat, S)

    return student_ts, diff_ts, disc_g


def kernel(student_id, exercise_id, S, theta_tuda, theta, diff_table,
           disc_table):
    sid2 = student_id.reshape(B // CH, CH)
    eid2 = exercise_id.reshape(B // CH, CH)
    theta_flat = jnp.pad(theta, ((0, NPAD - NTAB), (0, 0))).reshape(-1)
    disc_flat = jnp.pad(disc_table, ((0, NPAD - NTAB), (0, 0))).reshape(-1)
    student_ts, diff_ts, disc_rows = _run(sid2, eid2, S, theta_flat,
                                          disc_flat, diff_table)
    return student_ts, diff_ts, disc_rows.reshape(B, 1)


# speedup vs baseline: 3.7189x; 1.1192x over previous
import functools

import jax
import jax.numpy as jnp
from jax import lax
from jax.experimental import pallas as pl
from jax.experimental.pallas import tpu as pltpu
from jax.experimental.pallas import tpu_sc as plsc

B = 16384
K = 128
NC = 2
NS = 16
NW = NC * NS
ROWS_PER_W = B // NW
CH = 128
NCH = ROWS_PER_W // CH
NTAB = 1000000
NPAD = 1000448


def _sc_prime(sid_hbm, sid_out, sid_v, sem):
    wid = lax.axis_index("s") * NC + lax.axis_index("c")
    cbase = wid * NCH
    pltpu.sync_copy(sid_hbm.at[pl.ds(cbase, NCH)], sid_v)
    pltpu.sync_copy(sid_v, sid_out.at[pl.ds(cbase, NCH)])


def _sc_diff(eid_hbm, diff_hbm, diff_out, eid_v, dbuf, sem_d, sem_o):
    wid = lax.axis_index("s") * NC + lax.axis_index("c")
    cbase = wid * NCH
    pltpu.sync_copy(eid_hbm.at[pl.ds(cbase, NCH)], eid_v)
    hs = [pltpu.async_copy(diff_hbm.at[eid_v.at[j]], dbuf.at[j], sem_d)
          for j in range(NCH)]
    out_hs = []
    for j in range(NCH):
        hs[j].wait()
        out_hs.append(pltpu.async_copy(
            dbuf.at[j], diff_out.at[pl.ds((cbase + j) * CH, CH)], sem_o))
    for h in out_hs:
        h.wait()


def _sc_blend(sid_hbm, eid_hbm, theta_hbm, disc_hbm, s_hbm,
              student_out, disc_out,
              sid_v, eid_v, theta_v, disc_v, s_buf,
              sem_t, sem_c, sem_i, sem_s):
    wid = lax.axis_index("s") * NC + lax.axis_index("c")
    cbase = wid * NCH

    pltpu.sync_copy(sid_hbm.at[pl.ds(cbase, NCH)], sid_v)
    pltpu.sync_copy(eid_hbm.at[pl.ds(cbase, NCH)], eid_v)

    hs = []
    for j in range(NCH):
        hs.append(pltpu.async_copy(theta_hbm.at[sid_v.at[j]],
                                   theta_v.at[pl.ds(j * CH, CH)], sem_t))
        hs.append(pltpu.async_copy(disc_hbm.at[eid_v.at[j]], disc_v.at[j],
                                   sem_c))
    h_in0 = pltpu.async_copy(s_hbm.at[pl.ds(cbase * CH, CH)], s_buf.at[0],
                             sem_i)
    for h in hs:
        h.wait()
    pltpu.sync_copy(disc_v, disc_out.at[pl.ds(cbase, NCH)])

    h_in = h_in0
    h_out_prev = None
    for j in range(NCH):
        p = j % 2
        h_in.wait()
        if j + 1 < NCH:
            if h_out_prev is not None:
                h_out_prev.wait()
                h_out_prev = None
            h_in = pltpu.async_copy(
                s_hbm.at[pl.ds((cbase + j + 1) * CH, CH)], s_buf.at[1 - p],
                sem_i)

        def group_body(g, _, j=j, p=p):
            tv = theta_v[pl.ds(j * CH + g * 16, 16)]
            for l in range(16):
                t = tv.at[jnp.full((16,), l, dtype=jnp.int32)].get(
                    mode="promise_in_bounds")
                one_m_t = 1.0 - t
                r = g * 16 + l
                for c in range(K // 16):
                    sl = pl.ds(c * 16, 16)
                    s_buf[p, r, sl] = s_buf[p, r, sl] * one_m_t + t
            return 0

        lax.fori_loop(0, CH // 16, group_body, 0)
        h_out = pltpu.async_copy(
            s_buf.at[p], student_out.at[pl.ds((cbase + j) * CH, CH)], sem_s)
        if h_out_prev is not None:
            h_out_prev.wait()
        h_out_prev = h_out
    h_out_prev.wait()


@jax.jit
def _run(sid2, eid2, S, theta, disc_table, diff_table):
    mesh = plsc.VectorSubcoreMesh(core_axis_name="c", subcore_axis_name="s")
    sid2b, = pl.kernel(
        _sc_prime,
        out_type=[jax.ShapeDtypeStruct((B // CH, CH), jnp.int32)],
        mesh=mesh,
        scratch_types=[
            pltpu.VMEM((NCH, CH), jnp.int32),
            pltpu.SemaphoreType.DMA,
        ],
    )(sid2)

    diff_ts, = pl.kernel(
        _sc_diff,
        out_type=[jax.ShapeDtypeStruct((B, K), jnp.float32)],
        mesh=mesh,
        scratch_types=[
            pltpu.VMEM((NCH, CH), jnp.int32),
            pltpu.VMEM((NCH, CH, K), jnp.float32),
            pltpu.SemaphoreType.DMA,
            pltpu.SemaphoreType.DMA,
        ],
    )(eid2, diff_table)

    theta_b, disc_b, _ = jax.lax.optimization_barrier(
        (theta, disc_table, sid2b))
    theta_flat = jnp.pad(theta_b, ((0, NPAD - NTAB), (0, 0))).reshape(-1)
    disc_flat = jnp.pad(disc_b, ((0, NPAD - NTAB), (0, 0))).reshape(-1)

    student_ts, disc_g = pl.kernel(
        _sc_blend,
        out_type=[
            jax.ShapeDtypeStruct((B, K), jnp.float32),
            jax.ShapeDtypeStruct((B // CH, CH), jnp.float32),
        ],
        mesh=mesh,
        scratch_types=[
            pltpu.VMEM((NCH, CH), jnp.int32),
            pltpu.VMEM((NCH, CH), jnp.int32),
            pltpu.VMEM((ROWS_PER_W,), jnp.float32),
            pltpu.VMEM((NCH, CH), jnp.float32),
            pltpu.VMEM((2, CH, K), jnp.float32),
            pltpu.SemaphoreType.DMA,
            pltpu.SemaphoreType.DMA,
            pltpu.SemaphoreType.DMA,
            pltpu.SemaphoreType.DMA,
        ],
    )(sid2b, eid2, theta_flat, disc_flat, S)

    return student_ts, diff_ts, disc_g


def kernel(student_id, exercise_id, S, theta_tuda, theta, diff_table,
           disc_table):
    sid2 = student_id.reshape(B // CH, CH)
    eid2 = exercise_id.reshape(B // CH, CH)
    student_ts, diff_ts, disc_rows = _run(sid2, eid2, S, theta, disc_table,
                                          diff_table)
    return student_ts, diff_ts, disc_rows.reshape(B, 1)
